# baseline (speedup 1.0000x reference)
import functools

import jax
import jax.numpy as jnp
from jax import lax
from jax.experimental import pallas as pl
from jax.experimental.pallas import tpu as pltpu
from jax.experimental.pallas import tpu_sc as plsc

NC, NS, L = 2, 16, 16
NW = NC * NS

R, C = 16384, 200
ROWS_W = R // NW
RCHUNK = 64
NCHUNK = ROWS_W // RCHUNK
NBUF = 2
TAB = 128

COLS = list(range(0, C - L + 1, L))
if COLS[-1] + L < C:
    COLS.append(C - L)

_mesh = plsc.VectorSubcoreMesh(
    core_axis_name="c", subcore_axis_name="s", num_cores=NC, num_subcores=NS
)


@functools.partial(
    pl.kernel,
    out_type=jax.ShapeDtypeStruct((R, C), jnp.float32),
    mesh=_mesh,
    scratch_types=[
        pltpu.VMEM((TAB,), jnp.float32),
        pltpu.VMEM((RCHUNK, C), jnp.int32),
        pltpu.VMEM((RCHUNK, C), jnp.int32),
        pltpu.VMEM((RCHUNK, C), jnp.float32),
        pltpu.VMEM((RCHUNK, C), jnp.float32),
        pltpu.SemaphoreType.DMA,
        pltpu.SemaphoreType.DMA,
        pltpu.SemaphoreType.DMA,
        pltpu.SemaphoreType.DMA,
    ],
    compiler_params=pltpu.CompilerParams(needs_layout_passes=False),
)
def _lut_sc(
    idx_hbm, tab_hbm, out_hbm, tab_v, in0, in1, ob0, ob1, si0, si1, so0, so1
):
    ibufs = (in0, in1)
    obufs = (ob0, ob1)
    sem_in = (si0, si1)
    sem_out = (so0, so1)
    wid = lax.axis_index("s") * NC + lax.axis_index("c")
    base = wid * ROWS_W
    pltpu.sync_copy(tab_hbm, tab_v)

    in_cp = [None] * NBUF
    out_cp = [None] * NBUF
    in_cp[0] = pltpu.async_copy(
        idx_hbm.at[pl.ds(base, RCHUNK), :], ibufs[0], sem_in[0]
    )
    for c in range(NCHUNK):
        b = c % NBUF
        nb = (c + 1) % NBUF
        if c + 1 < NCHUNK:
            in_cp[nb] = pltpu.async_copy(
                idx_hbm.at[pl.ds(base + (c + 1) * RCHUNK, RCHUNK), :],
                ibufs[nb],
                sem_in[nb],
            )
        in_cp[b].wait()
        if out_cp[b] is not None:
            out_cp[b].wait()
            out_cp[b] = None
        ib, ob = ibufs[b], obufs[b]

        @plsc.parallel_loop(0, RCHUNK, step=1, unroll=4)
        def _(r):
            for col in COLS:
                iv = ib[r, pl.ds(col, L)]
                ob[r, pl.ds(col, L)] = plsc.load_gather(tab_v, [iv])

        out_cp[b] = pltpu.async_copy(
            obufs[b],
            out_hbm.at[pl.ds(base + c * RCHUNK, RCHUNK), :],
            sem_out[b],
        )
    for b in range(NBUF):
        if out_cp[b] is not None:
            out_cp[b].wait()


def kernel(idxs, labels):
    tab = jnp.zeros((TAB,), jnp.float32).at[: labels.shape[0]].set(labels)
    return _lut_sc(idxs.astype(jnp.int32), tab)


# speedup vs baseline: 1.0251x; 1.0251x over previous
import functools

import jax
import jax.numpy as jnp
from jax import lax
from jax.experimental import pallas as pl
from jax.experimental.pallas import tpu as pltpu
from jax.experimental.pallas import tpu_sc as plsc

NC, NS, L = 2, 16, 16
NW = NC * NS

R, C = 16384, 200
ROWS_W = R // NW
RCHUNK = 64
NCHUNK = ROWS_W // RCHUNK
NBUF = 2
TAB = 128

COLS = list(range(0, C - L + 1, L))
if COLS[-1] + L < C:
    COLS.append(C - L)

_mesh = plsc.VectorSubcoreMesh(
    core_axis_name="c", subcore_axis_name="s", num_cores=NC, num_subcores=NS
)


@functools.partial(
    pl.kernel,
    out_type=jax.ShapeDtypeStruct((R, C), jnp.float32),
    mesh=_mesh,
    scratch_types=[
        pltpu.VMEM((TAB,), jnp.float32),
        pltpu.VMEM((RCHUNK, C), jnp.int32),
        pltpu.VMEM((RCHUNK, C), jnp.int32),
        pltpu.VMEM((RCHUNK, C), jnp.float32),
        pltpu.VMEM((RCHUNK, C), jnp.float32),
        pltpu.SemaphoreType.DMA,
        pltpu.SemaphoreType.DMA,
        pltpu.SemaphoreType.DMA,
        pltpu.SemaphoreType.DMA,
    ],
    compiler_params=pltpu.CompilerParams(
        needs_layout_passes=False,
        skip_device_barrier=True,
        disable_bounds_checks=True,
        disable_semaphore_checks=True,
    ),
)
def _lut_sc(
    idx_hbm, tab_hbm, out_hbm, tab_v, in0, in1, ob0, ob1, si0, si1, so0, so1
):
    ibufs = (in0, in1)
    obufs = (ob0, ob1)
    sem_in = (si0, si1)
    sem_out = (so0, so1)
    wid = lax.axis_index("s") * NC + lax.axis_index("c")
    base = wid * ROWS_W
    pltpu.sync_copy(tab_hbm, tab_v)

    in_cp = [None] * NBUF
    out_cp = [None] * NBUF
    in_cp[0] = pltpu.async_copy(
        idx_hbm.at[pl.ds(base, RCHUNK), :], ibufs[0], sem_in[0]
    )
    for c in range(NCHUNK):
        b = c % NBUF
        nb = (c + 1) % NBUF
        if c + 1 < NCHUNK:
            in_cp[nb] = pltpu.async_copy(
                idx_hbm.at[pl.ds(base + (c + 1) * RCHUNK, RCHUNK), :],
                ibufs[nb],
                sem_in[nb],
            )
        in_cp[b].wait()
        if out_cp[b] is not None:
            out_cp[b].wait()
            out_cp[b] = None
        ib, ob = ibufs[b], obufs[b]

        @plsc.parallel_loop(0, RCHUNK, step=1, unroll=2)
        def _(r):
            for col in COLS:
                iv = ib[r, pl.ds(col, L)]
                ob[r, pl.ds(col, L)] = plsc.load_gather(tab_v, [iv])

        out_cp[b] = pltpu.async_copy(
            obufs[b],
            out_hbm.at[pl.ds(base + c * RCHUNK, RCHUNK), :],
            sem_out[b],
        )
    for b in range(NBUF):
        if out_cp[b] is not None:
            out_cp[b].wait()


def kernel(idxs, labels):
    tab = jnp.zeros((TAB,), jnp.float32).at[: labels.shape[0]].set(labels)
    return _lut_sc(idxs.astype(jnp.int32), tab)


# speedup vs baseline: 1.0438x; 1.0182x over previous
import functools

import jax
import jax.numpy as jnp
from jax import lax
from jax.experimental import pallas as pl
from jax.experimental.pallas import tpu as pltpu
from jax.experimental.pallas import tpu_sc as plsc

NC, NS, L = 2, 16, 16
NW = NC * NS

R, C = 16384, 200
ROWS_W = R // NW
RCHUNK = 64
NCHUNK = ROWS_W // RCHUNK
NBUF = 2
TAB = 128

COLS = list(range(0, C - L + 1, L))
if COLS[-1] + L < C:
    COLS.append(C - L)

_mesh = plsc.VectorSubcoreMesh(
    core_axis_name="c", subcore_axis_name="s", num_cores=NC, num_subcores=NS
)


@functools.partial(
    pl.kernel,
    out_type=jax.ShapeDtypeStruct((R, C), jnp.float32),
    mesh=_mesh,
    scratch_types=[
        pltpu.VMEM((TAB,), jnp.float32),
        pltpu.VMEM((RCHUNK, C), jnp.int32),
        pltpu.VMEM((RCHUNK, C), jnp.int32),
        pltpu.VMEM((RCHUNK, C), jnp.float32),
        pltpu.VMEM((RCHUNK, C), jnp.float32),
        pltpu.SemaphoreType.DMA,
        pltpu.SemaphoreType.DMA,
        pltpu.SemaphoreType.DMA,
        pltpu.SemaphoreType.DMA,
    ],
    compiler_params=pltpu.CompilerParams(needs_layout_passes=False),
)
def _lut_sc(
    idx_hbm, tab_hbm, out_hbm, tab_v, in0, in1, ob0, ob1, si0, si1, so0, so1
):
    ibufs = (in0, in1)
    obufs = (ob0, ob1)
    sem_in = (si0, si1)
    sem_out = (so0, so1)
    wid = lax.axis_index("s") * NC + lax.axis_index("c")
    base = wid * ROWS_W

    in_cp = [None] * NBUF
    out_cp = [None] * NBUF
    in_cp[0] = pltpu.async_copy(
        idx_hbm.at[pl.ds(base, RCHUNK), :], ibufs[0], sem_in[0]
    )
    pltpu.sync_copy(tab_hbm, tab_v)
    for c in range(NCHUNK):
        b = c % NBUF
        nb = (c + 1) % NBUF
        if c + 1 < NCHUNK:
            in_cp[nb] = pltpu.async_copy(
                idx_hbm.at[pl.ds(base + (c + 1) * RCHUNK, RCHUNK), :],
                ibufs[nb],
                sem_in[nb],
            )
        in_cp[b].wait()
        if out_cp[b] is not None:
            out_cp[b].wait()
            out_cp[b] = None
        ib, ob = ibufs[b], obufs[b]

        @plsc.parallel_loop(0, RCHUNK, step=1, unroll=2)
        def _(r):
            for col in COLS:
                iv = ib[r, pl.ds(col, L)]
                ob[r, pl.ds(col, L)] = plsc.load_gather(tab_v, [iv])

        out_cp[b] = pltpu.async_copy(
            obufs[b],
            out_hbm.at[pl.ds(base + c * RCHUNK, RCHUNK), :],
            sem_out[b],
        )
    for b in range(NBUF):
        if out_cp[b] is not None:
            out_cp[b].wait()


def kernel(idxs, labels):
    tab = jnp.zeros((TAB,), jnp.float32).at[: labels.shape[0]].set(labels)
    return _lut_sc(idxs.astype(jnp.int32), tab)
